# Initial kernel scaffold; baseline (speedup 1.0000x reference)
import jax
import jax.numpy as jnp
from jax import lax
from jax.experimental import pallas as pl
from jax.experimental.pallas import tpu as pltpu
from jax.experimental.pallas import tpu_sc as plsc

NC = 2
NS = 16
C = 128


def _seg_sum_partials(table, srcs, dsts, acc_rows):
  n, d = table.shape
  nw, kc, _ = srcs.shape
  rpw = acc_rows // NS
  zch = rpw // C

  mesh = plsc.VectorSubcoreMesh(core_axis_name="c", subcore_axis_name="s")

  def body(tbl_hbm, src_hbm, dst_hbm, zero_hbm, out_hbm,
           src_v, dst_v, rows_v, zrows_v, acc):
    c = lax.axis_index("c")
    s = lax.axis_index("s")
    wid = s * NC + c

    pltpu.sync_copy(zero_hbm, zrows_v)
    for z in range(zch):
      pltpu.sync_copy(zrows_v, acc.at[pl.ds((s * zch + z) * C, C)])
    plsc.subcore_barrier()

    pltpu.sync_copy(src_hbm.at[wid], src_v)
    pltpu.sync_copy(dst_hbm.at[wid], dst_v)

    def step(j, carry):
      pltpu.sync_copy(tbl_hbm.at[src_v.at[j]], rows_v)
      pltpu.sync_copy(rows_v, acc.at[dst_v.at[j]], add=True)
      return carry

    lax.fori_loop(0, kc, step, 0)
    plsc.subcore_barrier()

    pltpu.sync_copy(acc.at[pl.ds(s * rpw, rpw)],
                    out_hbm.at[c, pl.ds(s * rpw, rpw)])

  zeros = jnp.zeros((C, d), jnp.float32)
  return pl.kernel(
      body,
      out_type=jax.ShapeDtypeStruct((NC, acc_rows, d), jnp.float32),
      mesh=mesh,
      scratch_types=[
          pltpu.VMEM((kc, C), jnp.int32),
          pltpu.VMEM((kc, C), jnp.int32),
          pltpu.VMEM((C, d), jnp.float32),
          pltpu.VMEM((C, d), jnp.float32),
          pltpu.VMEM_SHARED((acc_rows, d), jnp.float32),
      ],
  )(table, srcs, dsts, zeros)


def _linear(p, w, b, relu, n):
  _, rows, d = p.shape
  dout = w.shape[1]
  blk = 2000
  assert n % blk == 0

  def body(p_ref, w_ref, b_ref, o_ref):
    ssum = p_ref[0] + p_ref[1]
    y = lax.dot_general(ssum, w_ref[...], (((1,), (0,)), ((), ())),
                        preferred_element_type=jnp.float32,
                        precision=lax.Precision.HIGHEST)
    y = y + b_ref[...]
    if relu:
      y = jnp.maximum(y, 0.0)
    o_ref[...] = y

  return pl.pallas_call(
      body,
      grid=(n // blk,),
      in_specs=[
          pl.BlockSpec((2, blk, d), lambda i: (0, i, 0)),
          pl.BlockSpec((d, dout), lambda i: (0, 0)),
          pl.BlockSpec((1, dout), lambda i: (0, 0)),
      ],
      out_specs=pl.BlockSpec((blk, dout), lambda i: (i, 0)),
      out_shape=jax.ShapeDtypeStruct((n, dout), jnp.float32),
  )(p, w, b.reshape(1, dout))


def kernel(x, edge_index, W1, b1, W2, b2):
  n, d = x.shape
  e = edge_index.shape[1]
  nw = NC * NS
  src = edge_index[0].astype(jnp.int32)
  dst = edge_index[1].astype(jnp.int32)

  kc = -(-e // (nw * C))
  e_pad = kc * nw * C
  acc_rows = (n // (NS * C) + 1) * NS * C

  pad = e_pad - e
  src_p = jnp.concatenate([src, jnp.zeros((pad,), jnp.int32)])
  dst_p = jnp.concatenate([dst, jnp.full((pad,), n, jnp.int32)])
  srcs = src_p.reshape(nw, kc, C)
  dsts = dst_p.reshape(nw, kc, C)

  p1 = _seg_sum_partials(x, srcs, dsts, acc_rows)
  h = _linear(p1, W1, b1, True, n)
  p2 = _seg_sum_partials(h, srcs, dsts, acc_rows)
  return _linear(p2, W2, b2, False, n)


# speedup vs baseline: 5.7980x; 5.7980x over previous
import jax
import jax.numpy as jnp
from jax import lax
from jax.experimental import pallas as pl
from jax.experimental.pallas import tpu as pltpu
from jax.experimental.pallas import tpu_sc as plsc

NC = 2
NS = 16
C = 128


def _seg_sum_split(table, srcs, dsts, acc_rows):
  _, dh = table.shape
  _, _, kc, _ = srcs.shape
  rpw = acc_rows // NS
  zch = rpw // C

  mesh = plsc.VectorSubcoreMesh(core_axis_name="c", subcore_axis_name="s")

  def body(tbl_hbm, src_hbm, dst_hbm, zero_hbm, out_hbm,
           src_v, dst_v, rows_v, zrows_v, acc):
    c = lax.axis_index("c")
    s = lax.axis_index("s")

    pltpu.sync_copy(zero_hbm, zrows_v)
    for z in range(zch):
      pltpu.sync_copy(zrows_v, acc.at[pl.ds((s * zch + z) * C, C)])
    plsc.subcore_barrier()

    pltpu.sync_copy(src_hbm.at[c, s], src_v)
    pltpu.sync_copy(dst_hbm.at[s], dst_v)

    def step(j, carry):
      pltpu.sync_copy(tbl_hbm.at[src_v.at[j]], rows_v)
      pltpu.sync_copy(rows_v, acc.at[dst_v.at[j]], add=True)
      return carry

    lax.fori_loop(0, kc, step, 0)
    plsc.subcore_barrier()

    pltpu.sync_copy(acc.at[pl.ds(s * rpw, rpw)],
                    out_hbm.at[c, pl.ds(s * rpw, rpw)])

  zeros = jnp.zeros((C, dh), jnp.float32)
  return pl.kernel(
      body,
      out_type=jax.ShapeDtypeStruct((NC, acc_rows, dh), jnp.float32),
      mesh=mesh,
      compiler_params=pltpu.CompilerParams(use_tc_tiling_on_sc=False),
      scratch_types=[
          pltpu.VMEM((kc, C), jnp.int32),
          pltpu.VMEM((kc, C), jnp.int32),
          pltpu.VMEM((C, dh), jnp.float32),
          pltpu.VMEM((C, dh), jnp.float32),
          pltpu.VMEM_SHARED((acc_rows, dh), jnp.float32),
      ],
  )(table, srcs, dsts, zeros)


def _linear(p, w, b, relu, split_out, n):
  _, rows, dh = p.shape
  dout = w.shape[1]
  blk = 2000
  assert n % blk == 0

  def body(p_ref, w_ref, b_ref, o_ref):
    ssum = jnp.concatenate([p_ref[0], p_ref[1]], axis=1)
    y = lax.dot_general(ssum, w_ref[...], (((1,), (0,)), ((), ())),
                        preferred_element_type=jnp.float32,
                        precision=lax.Precision.HIGHEST)
    y = y + b_ref[...]
    if relu:
      y = jnp.maximum(y, 0.0)
    if split_out:
      o_ref[0] = y[:, :dout // 2]
      o_ref[1] = y[:, dout // 2:]
    else:
      o_ref[...] = y

  if split_out:
    out_shape = jax.ShapeDtypeStruct((2, n, dout // 2), jnp.float32)
    out_specs = pl.BlockSpec((2, blk, dout // 2), lambda i: (0, i, 0))
  else:
    out_shape = jax.ShapeDtypeStruct((n, dout), jnp.float32)
    out_specs = pl.BlockSpec((blk, dout), lambda i: (i, 0))

  return pl.pallas_call(
      body,
      grid=(n // blk,),
      in_specs=[
          pl.BlockSpec((2, blk, dh), lambda i: (0, i, 0)),
          pl.BlockSpec((dh * 2, dout), lambda i: (0, 0)),
          pl.BlockSpec((1, dout), lambda i: (0, 0)),
      ],
      out_specs=out_specs,
      out_shape=out_shape,
  )(p, w, b.reshape(1, dout))


def kernel(x, edge_index, W1, b1, W2, b2):
  n, d = x.shape
  dh = d // 2
  e = edge_index.shape[1]
  src = edge_index[0].astype(jnp.int32)
  dst = edge_index[1].astype(jnp.int32)

  kc = -(-e // (NS * C))
  e_pad = kc * NS * C
  acc_rows = (n // (NS * C) + 1) * NS * C

  pad = e_pad - e
  src_p = jnp.concatenate([src, jnp.zeros((pad,), jnp.int32)])
  dst_p = jnp.concatenate([dst, jnp.full((pad,), n, jnp.int32)])
  srcs = src_p.reshape(NS, kc, C)
  srcs2 = jnp.stack([srcs, srcs + n])
  dsts = dst_p.reshape(NS, kc, C)

  x_flat = jnp.transpose(x.reshape(n, 2, dh), (1, 0, 2)).reshape(2 * n, dh)

  p1 = _seg_sum_split(x_flat, srcs2, dsts, acc_rows)
  h = _linear(p1, W1, b1, True, True, n)
  p2 = _seg_sum_split(h.reshape(2 * n, dh), srcs2, dsts, acc_rows)
  return _linear(p2, W2, b2, False, False, n)


# speedup vs baseline: 6.1152x; 1.0547x over previous
import jax
import jax.numpy as jnp
from jax import lax
from jax.experimental import pallas as pl
from jax.experimental.pallas import tpu as pltpu
from jax.experimental.pallas import tpu_sc as plsc

NC = 2
NS = 16
C = 128


def _seg_sum_split(table, srcs, dsts, acc_rows):
  _, dh = table.shape
  _, _, kc, _ = srcs.shape
  assert kc % 2 == 0
  rpw = acc_rows // NS
  zch = rpw // C

  mesh = plsc.VectorSubcoreMesh(core_axis_name="c", subcore_axis_name="s")

  def body(tbl_hbm, src_hbm, dst_hbm, zero_hbm, out_hbm,
           src_v, dst_v, r0, r1, zrows_v, acc, g0, g1, s0, s1):
    c = lax.axis_index("c")
    s = lax.axis_index("s")

    pltpu.sync_copy(zero_hbm, zrows_v)
    for z in range(zch):
      pltpu.sync_copy(zrows_v, acc.at[pl.ds((s * zch + z) * C, C)])
    plsc.subcore_barrier()

    pltpu.sync_copy(src_hbm.at[c, s], src_v)
    pltpu.sync_copy(dst_hbm.at[s], dst_v)

    def gather(j, buf, sem):
      return pltpu.async_copy(tbl_hbm.at[src_v.at[j]], buf, sem)

    def gather_wait(j, buf, sem):
      pltpu.make_async_copy(tbl_hbm.at[src_v.at[j]], buf, sem).wait()

    def scatter(j, buf, sem):
      return pltpu.async_copy(buf, acc.at[dst_v.at[j]], sem, add=True)

    def scatter_wait(j, buf, sem):
      pltpu.make_async_copy(buf, acc.at[dst_v.at[j]], sem).wait()

    gather(0, r0, g0)

    def pair(i, carry):
      j = 2 * i
      @pl.when(i > 0)
      def _():
        scatter_wait(j - 1, r1, s1)
      gather(j + 1, r1, g1)
      gather_wait(j, r0, g0)
      scatter(j, r0, s0)
      scatter_wait(j, r0, s0)
      @pl.when(i + 1 < kc // 2)
      def _():
        gather(j + 2, r0, g0)
      gather_wait(j + 1, r1, g1)
      scatter(j + 1, r1, s1)
      return carry

    lax.fori_loop(0, kc // 2, pair, 0)
    scatter_wait(kc - 1, r1, s1)
    plsc.subcore_barrier()

    pltpu.sync_copy(acc.at[pl.ds(s * rpw, rpw)],
                    out_hbm.at[c, pl.ds(s * rpw, rpw)])

  zeros = jnp.zeros((C, dh), jnp.float32)
  return pl.kernel(
      body,
      out_type=jax.ShapeDtypeStruct((NC, acc_rows, dh), jnp.float32),
      mesh=mesh,
      compiler_params=pltpu.CompilerParams(use_tc_tiling_on_sc=False),
      scratch_types=[
          pltpu.VMEM((kc, C), jnp.int32),
          pltpu.VMEM((kc, C), jnp.int32),
          pltpu.VMEM((C, dh), jnp.float32),
          pltpu.VMEM((C, dh), jnp.float32),
          pltpu.VMEM((C, dh), jnp.float32),
          pltpu.VMEM_SHARED((acc_rows, dh), jnp.float32),
          pltpu.SemaphoreType.DMA,
          pltpu.SemaphoreType.DMA,
          pltpu.SemaphoreType.DMA,
          pltpu.SemaphoreType.DMA,
      ],
  )(table, srcs, dsts, zeros)


def _linear(p, w, b, relu, n):
  _, rows, dh = p.shape
  dout = w.shape[1]
  blk = 2000
  assert n % blk == 0

  def body(p_ref, w_ref, b_ref, o_ref):
    ssum = jnp.concatenate([p_ref[0], p_ref[1]], axis=1)
    y = lax.dot_general(ssum, w_ref[...], (((1,), (0,)), ((), ())),
                        preferred_element_type=jnp.float32,
                        precision=lax.Precision.HIGHEST)
    y = y + b_ref[...]
    if relu:
      y = jnp.maximum(y, 0.0)
    o_ref[...] = y

  return pl.pallas_call(
      body,
      grid=(n // blk,),
      in_specs=[
          pl.BlockSpec((2, blk, dh), lambda i: (0, i, 0)),
          pl.BlockSpec((dh * 2, dout), lambda i: (0, 0)),
          pl.BlockSpec((1, dout), lambda i: (0, 0)),
      ],
      out_specs=pl.BlockSpec((blk, dout), lambda i: (i, 0)),
      out_shape=jax.ShapeDtypeStruct((n, dout), jnp.float32),
  )(p, w, b.reshape(1, dout))


def kernel(x, edge_index, W1, b1, W2, b2):
  n, d = x.shape
  dh = d // 2
  e = edge_index.shape[1]
  src = edge_index[0].astype(jnp.int32)
  dst = edge_index[1].astype(jnp.int32)

  kc = 2 * (-(-e // (NS * C * 2)))
  e_pad = kc * NS * C
  acc_rows = (n // (NS * C) + 1) * NS * C

  pad = e_pad - e
  src_p = jnp.concatenate([src, jnp.zeros((pad,), jnp.int32)])
  dst_p = jnp.concatenate([dst, jnp.full((pad,), n, jnp.int32)])
  srcs = src_p.reshape(NS, kc, C)
  srcs2 = jnp.stack([2 * srcs, 2 * srcs + 1])
  dsts = dst_p.reshape(NS, kc, C)

  p1 = _seg_sum_split(x.reshape(2 * n, dh), srcs2, dsts, acc_rows)
  h = _linear(p1, W1, b1, True, n)
  p2 = _seg_sum_split(h.reshape(2 * n, dh), srcs2, dsts, acc_rows)
  return _linear(p2, W2, b2, False, n)
